# Initial kernel scaffold; baseline (speedup 1.0000x reference)
import jax
import jax.numpy as jnp
from jax.experimental import pallas as pl


def kernel(x, node_in, node_out, relation, edge_weight, W_lin, b_lin, W_loop, b_loop):
    raise NotImplementedError("write your pallas kernel here")


# speedup vs baseline: 3.5627x; 3.5627x over previous
import functools

import jax
import jax.numpy as jnp
from jax import lax
from jax.experimental import pallas as pl
from jax.experimental.pallas import tpu as pltpu
from jax.experimental.pallas import tpu_sc as plsc

_N = 10000
_E = 320000
_D = 128
_R = 7
_O = 128
_K = 8

_NC = 2
_NS = 16
_NW = _NC * _NS
_EW = _E // _NW
_KC = 80
_NCHUNK = _EW // _KC
_NP = 10240
_RT = _NP // _NS
_RB = _KC


def _matmul_y(x, wt):
    bn = 1000

    def body(x_ref, w_ref, o_ref):
        o_ref[...] = jnp.dot(x_ref[...], w_ref[...],
                             preferred_element_type=jnp.float32)

    return pl.pallas_call(
        body,
        grid=(_N // bn,),
        in_specs=[
            pl.BlockSpec((bn, _D), lambda i: (i, 0)),
            pl.BlockSpec((_D, _K * _O), lambda i: (0, 0)),
        ],
        out_specs=pl.BlockSpec((bn, _K * _O), lambda i: (i, 0)),
        out_shape=jax.ShapeDtypeStruct((_N, _K * _O), jnp.float32),
    )(x, wt)


def _make_sc_scatter():
    mesh = plsc.VectorSubcoreMesh(core_axis_name="c", subcore_axis_name="s")

    @functools.partial(
        pl.kernel,
        mesh=mesh,
        out_type=jax.ShapeDtypeStruct((_NC, _NP, _O), jnp.float32),
        scratch_types=[
            pltpu.VMEM((_KC,), jnp.int32),
            pltpu.VMEM((_KC,), jnp.int32),
            pltpu.VMEM((_KC,), jnp.int32),
            pltpu.VMEM((_KC,), jnp.float32),
            pltpu.VMEM((_KC,), jnp.int32),
            pltpu.VMEM((_KC, _O), jnp.float32),
            pltpu.VMEM_SHARED((_NP, _O), jnp.float32),
            pltpu.SemaphoreType.DMA,
        ],
    )
    def sc_scatter(ni_hbm, rel_hbm, no_hbm, ew_hbm, tbl_hbm, out_hbm,
                   ni_v, rel_v, nc_v, ew_v, gc_v, rows_v, acc, sem):
        c = lax.axis_index("c")
        s = lax.axis_index("s")
        w = s * _NC + c

        zv = jnp.zeros((16,), jnp.float32)

        def zbody(i, carry):
            for j in range(_O // 16):
                rows_v[i, pl.ds(j * 16, 16)] = zv
            return carry

        lax.fori_loop(0, _RB, zbody, 0)
        for t in range(_RT // _RB):
            pltpu.sync_copy(rows_v, acc.at[pl.ds(s * _RT + t * _RB, _RB)])
        plsc.subcore_barrier()

        base = w * _EW
        gdn = lax.GatherDimensionNumbers(
            offset_dims=(), collapsed_slice_dims=(0,), start_index_map=(0,))

        def chunk_body(t, carry):
            eb = base + t * _KC
            pltpu.sync_copy(ni_hbm.at[pl.ds(eb, _KC)], ni_v)
            pltpu.sync_copy(rel_hbm.at[pl.ds(eb, _KC)], rel_v)
            pltpu.sync_copy(no_hbm.at[pl.ds(eb, _KC)], nc_v)
            pltpu.sync_copy(ew_hbm.at[pl.ds(eb, _KC)], ew_v)
            for g in range(_KC // 16):
                dl = pl.ds(g * 16, 16)
                gc_v[dl] = ni_v[dl] * _K + rel_v[dl]
            pltpu.async_copy(tbl_hbm.at[gc_v], rows_v, sem).wait()
            for g in range(_KC // 16):
                ewg = ew_v[pl.ds(g * 16, 16)]
                for l in range(16):
                    e = g * 16 + l
                    sp = lax.gather(
                        ewg, jnp.full((16, 1), l, jnp.int32), gdn,
                        slice_sizes=(1,),
                        mode=lax.GatherScatterMode.PROMISE_IN_BOUNDS)
                    for j in range(_O // 16):
                        sl2 = pl.ds(j * 16, 16)
                        rows_v[e, sl2] = rows_v[e, sl2] * sp
            pltpu.sync_copy(rows_v, acc.at[nc_v], add=True)
            return carry

        lax.fori_loop(0, _NCHUNK, chunk_body, 0)

        plsc.subcore_barrier()
        for t in range(_RT // _RB):
            r0 = s * _RT + t * _RB
            pltpu.sync_copy(acc.at[pl.ds(r0, _RB)], rows_v)
            pltpu.sync_copy(rows_v, out_hbm.at[c, pl.ds(r0, _RB)])

    return sc_scatter


_sc_scatter = _make_sc_scatter()


def _finish(psum, yloop, bias):
    bn = 1000

    def body(p_ref, y_ref, b_ref, o_ref):
        o_ref[...] = jnp.maximum(
            p_ref[0] + p_ref[1] + y_ref[...] + b_ref[...], 0.0)

    return pl.pallas_call(
        body,
        grid=(_N // bn,),
        in_specs=[
            pl.BlockSpec((_NC, bn, _O), lambda i: (0, i, 0)),
            pl.BlockSpec((bn, _O), lambda i: (i, 0)),
            pl.BlockSpec((1, _O), lambda i: (0, 0)),
        ],
        out_specs=pl.BlockSpec((bn, _O), lambda i: (i, 0)),
        out_shape=jax.ShapeDtypeStruct((_N, _O), jnp.float32),
    )(psum, yloop, bias)


def kernel(x, node_in, node_out, relation, edge_weight,
           W_lin, b_lin, W_loop, b_loop):
    ni = node_in.astype(jnp.int32)
    no = node_out.astype(jnp.int32)
    rel = relation.astype(jnp.int32)
    ew = edge_weight.astype(jnp.float32)

    wb = jnp.concatenate(
        [W_lin.reshape(_O, _R, _D).transpose(1, 0, 2), W_loop[None]], axis=0)
    wt = wb.reshape(_K * _O, _D).T

    y = _matmul_y(x, wt)
    tbl = y.reshape(_N * _K, _O)
    psum = _sc_scatter(ni, rel, no, ew, tbl)
    yloop = y[:, _R * _O:]
    bias = (b_lin + b_loop).reshape(1, _O)
    return _finish(psum, yloop, bias)


# speedup vs baseline: 7.8074x; 2.1914x over previous
import functools

import jax
import jax.numpy as jnp
from jax import lax
from jax.experimental import pallas as pl
from jax.experimental.pallas import tpu as pltpu
from jax.experimental.pallas import tpu_sc as plsc

_N = 10000
_E = 320000
_D = 128
_R = 7
_O = 128
_K = 8

_NC = 2
_NS = 16
_NW = _NC * _NS
_EW = _E // _NW
_KC = 80
_NCHUNK = _EW // _KC
_NP = 10240
_RT = _NP // _NS
_RB = _KC
_NPAIR = 62


def _matmul_y(x, wt):
    bn = 1000

    def body(x_ref, w_ref, o_ref):
        o_ref[...] = jnp.dot(x_ref[...], w_ref[...],
                             preferred_element_type=jnp.float32)

    return pl.pallas_call(
        body,
        grid=(_N // bn,),
        in_specs=[
            pl.BlockSpec((bn, _D), lambda i: (i, 0)),
            pl.BlockSpec((_D, _K * _O), lambda i: (0, 0)),
        ],
        out_specs=pl.BlockSpec((bn, _K * _O), lambda i: (i, 0)),
        out_shape=jax.ShapeDtypeStruct((_N, _K * _O), jnp.float32),
    )(x, wt)


def _make_sc_scatter():
    mesh = plsc.VectorSubcoreMesh(core_axis_name="c", subcore_axis_name="s")

    @functools.partial(
        pl.kernel,
        mesh=mesh,
        out_type=jax.ShapeDtypeStruct((_NC, _NP, _O), jnp.float32),
        scratch_types=[
            pltpu.VMEM((2, _KC), jnp.int32),
            pltpu.VMEM((2, _KC), jnp.int32),
            pltpu.VMEM((2, _KC), jnp.int32),
            pltpu.VMEM((2, _KC), jnp.float32),
            pltpu.VMEM((2, _KC), jnp.int32),
            pltpu.VMEM((2, _KC, _O), jnp.float32),
            pltpu.VMEM_SHARED((_NP, _O), jnp.float32),
            pltpu.SemaphoreType.DMA,
            pltpu.SemaphoreType.DMA,
            pltpu.SemaphoreType.DMA,
            pltpu.SemaphoreType.DMA,
            pltpu.SemaphoreType.DMA,
            pltpu.SemaphoreType.DMA,
            pltpu.SemaphoreType.DMA,
            pltpu.SemaphoreType.DMA,
        ],
    )
    def sc_scatter(ni_hbm, rel_hbm, no_hbm, ew_hbm, tbl_hbm, out_hbm,
                   ni_v, rel_v, nc_v, ew_v, gc_v, rows_v, acc,
                   sm0, sm1, sn0, sn1, sg0, sg1, ss0, ss1):
        c = lax.axis_index("c")
        s = lax.axis_index("s")
        w = s * _NC + c
        sm = (sm0, sm1)
        sn = (sn0, sn1)
        sg = (sg0, sg1)
        ss = (ss0, ss1)

        zv = jnp.zeros((16,), jnp.float32)

        def zbody(i, carry):
            for j in range(_O // 16):
                rows_v[0, i, pl.ds(j * 16, 16)] = zv
            return carry

        lax.fori_loop(0, _RB, zbody, 0)
        for t in range(_RT // _RB):
            pltpu.sync_copy(rows_v.at[0], acc.at[pl.ds(s * _RT + t * _RB, _RB)])
        plsc.subcore_barrier()

        base = w * _EW
        gdn = lax.GatherDimensionNumbers(
            offset_dims=(), collapsed_slice_dims=(0,), start_index_map=(0,))

        def meta_start(ck, b):
            eb = base + ck * _KC
            pltpu.async_copy(ni_hbm.at[pl.ds(eb, _KC)], ni_v.at[b], sm[b])
            pltpu.async_copy(rel_hbm.at[pl.ds(eb, _KC)], rel_v.at[b], sm[b])
            pltpu.async_copy(ew_hbm.at[pl.ds(eb, _KC)], ew_v.at[b], sm[b])

        def meta_wait(ck, b):
            eb = base + ck * _KC
            pltpu.make_async_copy(
                ni_hbm.at[pl.ds(eb, _KC)], ni_v.at[b], sm[b]).wait()
            pltpu.make_async_copy(
                rel_hbm.at[pl.ds(eb, _KC)], rel_v.at[b], sm[b]).wait()
            pltpu.make_async_copy(
                ew_hbm.at[pl.ds(eb, _KC)], ew_v.at[b], sm[b]).wait()

        def gc_build(b):
            for g in range(_KC // 16):
                dl = pl.ds(g * 16, 16)
                gc_v[b, dl] = ni_v[b, dl] * _K + rel_v[b, dl]

        def scale(b):
            for g in range(_KC // 16):
                ewg = ew_v[b, pl.ds(g * 16, 16)]
                for l in range(16):
                    e = g * 16 + l
                    sp = lax.gather(
                        ewg, jnp.full((16, 1), l, jnp.int32), gdn,
                        slice_sizes=(1,),
                        mode=lax.GatherScatterMode.PROMISE_IN_BOUNDS)
                    for j in range(_O // 16):
                        sl2 = pl.ds(j * 16, 16)
                        rows_v[b, e, sl2] = rows_v[b, e, sl2] * sp

        for b in range(2):
            meta_start(jnp.int32(b), b)

        def pair_body(k, carry):
            for b in range(2):
                ck = k * 2 + b
                meta_wait(ck, b)
                gc_build(b)

                @pl.when(k > 0)
                def _():
                    pltpu.make_async_copy(
                        rows_v.at[b], acc.at[nc_v.at[b]], ss[b]).wait()

                eb = base + ck * _KC
                pltpu.async_copy(no_hbm.at[pl.ds(eb, _KC)], nc_v.at[b], sn[b])
                pltpu.async_copy(tbl_hbm.at[gc_v.at[b]], rows_v.at[b], sg[b])
            for b in range(2):
                ck = k * 2 + b
                pltpu.make_async_copy(
                    tbl_hbm.at[gc_v.at[b]], rows_v.at[b], sg[b]).wait()
                scale(b)
                eb = base + ck * _KC
                pltpu.make_async_copy(
                    no_hbm.at[pl.ds(eb, _KC)], nc_v.at[b], sn[b]).wait()
                pltpu.async_copy(
                    rows_v.at[b], acc.at[nc_v.at[b]], ss[b], add=True)

                @pl.when(k < _NPAIR - 1)
                def _():
                    meta_start(ck + 2, b)
            return carry

        lax.fori_loop(0, _NPAIR, pair_body, 0)

        for b in range(2):
            pltpu.make_async_copy(
                rows_v.at[b], acc.at[nc_v.at[b]], ss[b]).wait()
        eb = base + (_NCHUNK - 1) * _KC
        pltpu.sync_copy(ni_hbm.at[pl.ds(eb, _KC)], ni_v.at[0])
        pltpu.sync_copy(rel_hbm.at[pl.ds(eb, _KC)], rel_v.at[0])
        pltpu.sync_copy(no_hbm.at[pl.ds(eb, _KC)], nc_v.at[0])
        pltpu.sync_copy(ew_hbm.at[pl.ds(eb, _KC)], ew_v.at[0])
        gc_build(0)
        pltpu.async_copy(tbl_hbm.at[gc_v.at[0]], rows_v.at[0], sg[0]).wait()
        scale(0)
        pltpu.sync_copy(rows_v.at[0], acc.at[nc_v.at[0]], add=True)

        plsc.subcore_barrier()
        for t in range(_RT // _RB):
            r0 = s * _RT + t * _RB
            pltpu.sync_copy(acc.at[pl.ds(r0, _RB)], rows_v.at[0])
            pltpu.sync_copy(rows_v.at[0], out_hbm.at[c, pl.ds(r0, _RB)])

    return sc_scatter


_sc_scatter = _make_sc_scatter()


def _finish(psum, yloop, bias):
    bn = 1000

    def body(p_ref, y_ref, b_ref, o_ref):
        o_ref[...] = jnp.maximum(
            p_ref[0] + p_ref[1] + y_ref[...] + b_ref[...], 0.0)

    return pl.pallas_call(
        body,
        grid=(_N // bn,),
        in_specs=[
            pl.BlockSpec((_NC, bn, _O), lambda i: (0, i, 0)),
            pl.BlockSpec((bn, _O), lambda i: (i, 0)),
            pl.BlockSpec((1, _O), lambda i: (0, 0)),
        ],
        out_specs=pl.BlockSpec((bn, _O), lambda i: (i, 0)),
        out_shape=jax.ShapeDtypeStruct((_N, _O), jnp.float32),
    )(psum, yloop, bias)


def kernel(x, node_in, node_out, relation, edge_weight,
           W_lin, b_lin, W_loop, b_loop):
    ni = node_in.astype(jnp.int32)
    no = node_out.astype(jnp.int32)
    rel = relation.astype(jnp.int32)
    ew = edge_weight.astype(jnp.float32)

    wb = jnp.concatenate(
        [W_lin.reshape(_O, _R, _D).transpose(1, 0, 2), W_loop[None]], axis=0)
    wt = wb.reshape(_K * _O, _D).T

    y = _matmul_y(x, wt)
    tbl = y.reshape(_N * _K, _O)
    psum = _sc_scatter(ni, rel, no, ew, tbl)
    yloop = y[:, _R * _O:]
    bias = (b_lin + b_loop).reshape(1, _O)
    return _finish(psum, yloop, bias)
